# Initial kernel scaffold; baseline (speedup 1.0000x reference)
import jax
import jax.numpy as jnp
from jax.experimental import pallas as pl


def kernel(batch_size, num_heads, seq_len, attention_scores):
    raise NotImplementedError("write your pallas kernel here")


# speedup vs baseline: 112.7001x; 112.7001x over previous
import functools

import jax
import jax.numpy as jnp
import numpy as np
from jax.experimental import pallas as pl
from jax.experimental.pallas import tpu as pltpu

_SPARSITY_RATIO = 0.9
_THRESHOLD_PERCENTILE = 0.95


def _sortable_to_f32(u):
    bits = jnp.where(
        u >= jnp.uint32(0x80000000), u ^ jnp.uint32(0x80000000), ~u
    )
    return jax.lax.bitcast_convert_type(bits, jnp.float32)


def _select_kernel(x_ref, thr_ref, cnt_ref, *, k_rank, n, row_chunks):
    rows = x_ref.shape[1]
    chunk = rows // row_chunks

    def count_ge(f):
        parts = [
            jnp.sum((x_ref[0, c * chunk:(c + 1) * chunk, :] >= f)
                    .astype(jnp.float32))
            for c in range(row_chunks)
        ]
        total = parts[0]
        for p in parts[1:]:
            total = total + p
        return total

    k_f = jnp.float32(k_rank)

    def body(_, carry):
        lo, hi, cnt_lo = carry
        mid = lo + ((hi - lo + jnp.uint32(1)) >> jnp.uint32(1))
        c = count_ge(_sortable_to_f32(mid))
        pred = c >= k_f
        lo2 = jnp.where(pred, mid, lo)
        hi2 = jnp.where(pred, hi, mid - jnp.uint32(1))
        cnt2 = jnp.where(pred, c, cnt_lo)
        return lo2, hi2, cnt2

    lo0 = jnp.uint32(0x007FFFFF)
    hi0 = jnp.uint32(0xFF800000)
    cnt0 = jnp.float32(n)
    lo, _, cnt = jax.lax.fori_loop(0, 32, body, (lo0, hi0, cnt0))
    pid = pl.program_id(0)
    thr_ref[pid] = _sortable_to_f32(lo)
    cnt_ref[pid] = cnt.astype(jnp.int32)


def _mask_kernel(thr_ref, x_ref, mask_ref):
    h = pl.program_id(0)
    mask_ref[0] = x_ref[0] >= thr_ref[h]


def kernel(batch_size, num_heads, seq_len, attention_scores):
    B, H, Sq, Skv = attention_scores.shape
    n = Sq * Skv
    BH = B * H
    x = attention_scores.reshape(BH, Sq, Skv)

    loc = np.float32(_THRESHOLD_PERCENTILE) * np.float32(n - 1)
    idx_hi = int(np.ceil(np.float64(loc)))
    k_rank = max(1, n - idx_hi)

    thr, counts = pl.pallas_call(
        functools.partial(_select_kernel, k_rank=k_rank, n=n, row_chunks=8),
        grid=(BH,),
        in_specs=[
            pl.BlockSpec((1, Sq, Skv), lambda i: (i, 0, 0)),
        ],
        out_specs=[
            pl.BlockSpec((BH,), lambda i: (0,), memory_space=pltpu.SMEM),
            pl.BlockSpec((BH,), lambda i: (0,), memory_space=pltpu.SMEM),
        ],
        out_shape=[
            jax.ShapeDtypeStruct((BH,), jnp.float32),
            jax.ShapeDtypeStruct((BH,), jnp.int32),
        ],
        compiler_params=pltpu.CompilerParams(
            dimension_semantics=("arbitrary",),
            vmem_limit_bytes=60 * 1024 * 1024,
        ),
    )(x)

    row_blk = 256
    mask3 = pl.pallas_call(
        _mask_kernel,
        grid=(BH, Sq // row_blk),
        in_specs=[
            pl.BlockSpec((BH,), lambda i, j: (0,), memory_space=pltpu.SMEM),
            pl.BlockSpec((1, row_blk, Skv), lambda i, j: (i, j, 0)),
        ],
        out_specs=pl.BlockSpec((1, row_blk, Skv), lambda i, j: (i, j, 0)),
        out_shape=jax.ShapeDtypeStruct((BH, Sq, Skv), jnp.bool_),
        compiler_params=pltpu.CompilerParams(
            dimension_semantics=("arbitrary", "arbitrary"),
            vmem_limit_bytes=60 * 1024 * 1024,
        ),
    )(thr, x)

    mask = mask3.reshape(B, H, Sq, Skv)
    density = jnp.sum(counts).astype(jnp.float32) / np.float32(BH * n)

    k = max(1, int(Skv * (1.0 - _SPARSITY_RATIO)))

    def topk_branch():
        _, topk_idx = jax.lax.top_k(attention_scores, k)
        bidx = jnp.arange(B)[:, None, None, None]
        hidx = jnp.arange(H)[None, :, None, None]
        qidx = jnp.arange(Sq)[None, None, :, None]
        topk_mask = jnp.zeros((B, H, Sq, Skv), dtype=bool)
        return topk_mask.at[bidx, hidx, qidx, topk_idx].set(True)

    return jax.lax.cond(
        density > np.float32(1.0 - _SPARSITY_RATIO),
        topk_branch,
        lambda: mask,
    )
